# baseline (speedup 1.0000x reference)
import jax
import jax.numpy as jnp
from jax import lax
from jax.experimental import pallas as pl
from jax.experimental.pallas import tpu as pltpu
from jax.experimental.pallas import tpu_sc as plsc

N = 100000
DIM_H = 512
NUM_GRAPHS = 256
ROWS = 1000
NB = N // ROWS

NW = 32
CHUNK = 112
TRIPS = 7


def _pool_ffn_body(batch_ref, h_ref, W1_ref, b1_ref, W2_ref, b2_ref,
                   out_ref, acc_ref):
    i = pl.program_id(0)

    @pl.when(i == 0)
    def _init():
        acc_ref[...] = jnp.zeros_like(acc_ref)

    ids = batch_ref[0, 0, :]
    seg = lax.broadcasted_iota(jnp.int32, (NUM_GRAPHS, ROWS), 0)
    onehot = (ids[None, :] == seg).astype(jnp.float32)
    acc_ref[...] += jnp.dot(onehot, h_ref[...],
                            preferred_element_type=jnp.float32)

    @pl.when(i == NB - 1)
    def _ffn():
        s = acc_ref[...]
        z = jnp.maximum(jnp.dot(s, W1_ref[...],
                                preferred_element_type=jnp.float32)
                        + b1_ref[...], 0.0)
        out_ref[...] = jnp.dot(z, W2_ref[...],
                               preferred_element_type=jnp.float32) + b2_ref[...]


def _broadcast_body(batch_ref, vn_ref, out_ref):
    ids = batch_ref[0, 0, :]
    seg = lax.broadcasted_iota(jnp.int32, (ROWS, NUM_GRAPHS), 1)
    onehot = (ids[:, None] == seg).astype(jnp.float32)
    out_ref[...] = jnp.dot(onehot, vn_ref[...],
                           preferred_element_type=jnp.float32)


def _sc_stream_body(h_hbm, out_hbm, buf0, buf1, sem0, sem1):
    cid = lax.axis_index("c")
    sid = lax.axis_index("s")
    wid = cid * 16 + sid
    bufs = (buf0, buf1)
    sems = (sem0, sem1)

    def desc(t):
        start = 75000 + (wid + t * NW) * CHUNK
        start = jnp.minimum(start, N - CHUNK)
        return pltpu.make_async_copy(h_hbm.at[pl.ds(start, CHUNK)],
                                     bufs[t % 2], sems[t % 2])

    desc(0).start()
    for t in range(TRIPS):
        desc(t).wait()
        if t + 1 < TRIPS:
            desc(t + 1).start()

    @pl.when(wid == 0)
    def _emit():
        pltpu.sync_copy(buf0.at[pl.ds(0, 8)], out_hbm)


@jax.jit
def kernel(h, batch, W1, b1, W2, b2):
    batch3 = batch.astype(jnp.int32).reshape(NB, 1, ROWS)

    sc_stream = pl.kernel(
        _sc_stream_body,
        out_type=jax.ShapeDtypeStruct((8, DIM_H), jnp.float32),
        mesh=plsc.VectorSubcoreMesh(core_axis_name="c", subcore_axis_name="s"),
        scratch_types=[
            pltpu.VMEM((CHUNK, DIM_H), jnp.float32),
            pltpu.VMEM((CHUNK, DIM_H), jnp.float32),
            pltpu.SemaphoreType.DMA,
            pltpu.SemaphoreType.DMA,
        ],
    )
    junk = sc_stream(h)

    h_vn = pl.pallas_call(
        _pool_ffn_body,
        grid=(NB,),
        in_specs=[
            pl.BlockSpec((1, 1, ROWS), lambda i: (i, 0, 0)),
            pl.BlockSpec((ROWS, DIM_H), lambda i: (i, 0)),
            pl.BlockSpec((DIM_H, 2 * DIM_H), lambda i: (0, 0)),
            pl.BlockSpec((2 * DIM_H,), lambda i: (0,)),
            pl.BlockSpec((2 * DIM_H, DIM_H), lambda i: (0, 0)),
            pl.BlockSpec((DIM_H,), lambda i: (0,)),
        ],
        out_specs=pl.BlockSpec((NUM_GRAPHS, DIM_H), lambda i: (0, 0)),
        out_shape=jax.ShapeDtypeStruct((NUM_GRAPHS, DIM_H), jnp.float32),
        scratch_shapes=[pltpu.VMEM((NUM_GRAPHS, DIM_H), jnp.float32)],
    )(batch3, h, W1, b1, W2, b2)

    out = pl.pallas_call(
        _broadcast_body,
        grid=(NB,),
        in_specs=[
            pl.BlockSpec((1, 1, ROWS), lambda i: (i, 0, 0)),
            pl.BlockSpec((NUM_GRAPHS, DIM_H), lambda i: (0, 0)),
        ],
        out_specs=pl.BlockSpec((ROWS, DIM_H), lambda i: (i, 0)),
        out_shape=jax.ShapeDtypeStruct((N, DIM_H), jnp.float32),
    )(batch3, h_vn)
    return out.at[0, 0].add(junk[0, 0] * 1e-30)


# speedup vs baseline: 1.5804x; 1.5804x over previous
import jax
import jax.numpy as jnp
from jax import lax
from jax.experimental import pallas as pl
from jax.experimental.pallas import tpu as pltpu

N = 100000
DIM_H = 512
NUM_GRAPHS = 256
ROWS = 2000
NB = N // ROWS


def _fused_body(batch_ref, h_ref, W1_ref, b1_ref, W2_ref, b2_ref,
                out_ref, acc_ref, vn_ref):
    i = pl.program_id(0)

    @pl.when(i == 0)
    def _init():
        acc_ref[...] = jnp.zeros_like(acc_ref)

    @pl.when(i < NB)
    def _pool():
        ids = batch_ref[0, 0, :]
        seg = lax.broadcasted_iota(jnp.int32, (NUM_GRAPHS, ROWS), 0)
        onehot = (ids[None, :] == seg).astype(jnp.float32)
        acc_ref[...] += jnp.dot(onehot, h_ref[...],
                                preferred_element_type=jnp.float32)

    @pl.when(i == NB - 1)
    def _ffn():
        s = acc_ref[...]
        z = jnp.maximum(jnp.dot(s, W1_ref[...],
                                preferred_element_type=jnp.float32)
                        + b1_ref[...], 0.0)
        vn_ref[...] = jnp.dot(z, W2_ref[...],
                              preferred_element_type=jnp.float32) + b2_ref[...]

    @pl.when(i >= NB)
    def _broadcast():
        ids = batch_ref[0, 0, :]
        seg = lax.broadcasted_iota(jnp.int32, (ROWS, NUM_GRAPHS), 1)
        onehot = (ids[:, None] == seg).astype(jnp.float32)
        out_ref[...] = jnp.dot(onehot, vn_ref[...],
                               preferred_element_type=jnp.float32)


@jax.jit
def kernel(h, batch, W1, b1, W2, b2):
    batch3 = batch.astype(jnp.int32).reshape(NB, 1, ROWS)

    out = pl.pallas_call(
        _fused_body,
        grid=(2 * NB,),
        in_specs=[
            pl.BlockSpec((1, 1, ROWS),
                         lambda i: (jnp.where(i < NB, i, i - NB), 0, 0)),
            pl.BlockSpec((ROWS, DIM_H),
                         lambda i: (jnp.minimum(i, NB - 1), 0)),
            pl.BlockSpec((DIM_H, 2 * DIM_H), lambda i: (0, 0)),
            pl.BlockSpec((2 * DIM_H,), lambda i: (0,)),
            pl.BlockSpec((2 * DIM_H, DIM_H), lambda i: (0, 0)),
            pl.BlockSpec((DIM_H,), lambda i: (0,)),
        ],
        out_specs=pl.BlockSpec((ROWS, DIM_H),
                               lambda i: (jnp.maximum(i - NB, 0), 0)),
        out_shape=jax.ShapeDtypeStruct((N, DIM_H), jnp.float32),
        scratch_shapes=[pltpu.VMEM((NUM_GRAPHS, DIM_H), jnp.float32),
                        pltpu.VMEM((NUM_GRAPHS, DIM_H), jnp.float32)],
    )(batch3, h, W1, b1, W2, b2)
    return out


# speedup vs baseline: 1.8357x; 1.1615x over previous
import jax
import jax.numpy as jnp
from jax import lax
from jax.experimental import pallas as pl
from jax.experimental.pallas import tpu as pltpu

N = 100000
DIM_H = 512
NUM_GRAPHS = 256
ROWS = 4000
NB = N // ROWS


def _fused_body(batch_ref, h_ref, W1_ref, b1_ref, W2_ref, b2_ref,
                out_ref, acc_ref, vn_ref):
    i = pl.program_id(0)

    @pl.when(i == 0)
    def _init():
        acc_ref[...] = jnp.zeros_like(acc_ref)

    @pl.when(i < NB)
    def _pool():
        ids = batch_ref[0, 0, :]
        seg = lax.broadcasted_iota(jnp.int32, (NUM_GRAPHS, ROWS), 0)
        onehot = (ids[None, :] == seg).astype(jnp.float32)
        acc_ref[...] += jnp.dot(onehot, h_ref[...],
                                preferred_element_type=jnp.float32)

    @pl.when(i == NB - 1)
    def _ffn():
        s = acc_ref[...]
        z = jnp.maximum(jnp.dot(s, W1_ref[...],
                                preferred_element_type=jnp.float32)
                        + b1_ref[...], 0.0)
        vn_ref[...] = jnp.dot(z, W2_ref[...],
                              preferred_element_type=jnp.float32) + b2_ref[...]

    @pl.when(i >= NB)
    def _broadcast():
        ids = batch_ref[0, 0, :]
        seg = lax.broadcasted_iota(jnp.int32, (ROWS, NUM_GRAPHS), 1)
        onehot = (ids[:, None] == seg).astype(jnp.float32)
        out_ref[...] = jnp.dot(onehot, vn_ref[...],
                               preferred_element_type=jnp.float32)


@jax.jit
def kernel(h, batch, W1, b1, W2, b2):
    batch3 = batch.astype(jnp.int32).reshape(NB, 1, ROWS)

    out = pl.pallas_call(
        _fused_body,
        grid=(2 * NB,),
        in_specs=[
            pl.BlockSpec((1, 1, ROWS),
                         lambda i: (jnp.where(i < NB, i, i - NB), 0, 0)),
            pl.BlockSpec((ROWS, DIM_H),
                         lambda i: (jnp.minimum(i, NB - 1), 0)),
            pl.BlockSpec((DIM_H, 2 * DIM_H), lambda i: (0, 0)),
            pl.BlockSpec((2 * DIM_H,), lambda i: (0,)),
            pl.BlockSpec((2 * DIM_H, DIM_H), lambda i: (0, 0)),
            pl.BlockSpec((DIM_H,), lambda i: (0,)),
        ],
        out_specs=pl.BlockSpec((ROWS, DIM_H),
                               lambda i: (jnp.maximum(i - NB, 0), 0)),
        out_shape=jax.ShapeDtypeStruct((N, DIM_H), jnp.float32),
        scratch_shapes=[pltpu.VMEM((NUM_GRAPHS, DIM_H), jnp.float32),
                        pltpu.VMEM((NUM_GRAPHS, DIM_H), jnp.float32)],
    )(batch3, h, W1, b1, W2, b2)
    return out


# speedup vs baseline: 1.8424x; 1.0037x over previous
import jax
import jax.numpy as jnp
from jax import lax
from jax.experimental import pallas as pl
from jax.experimental.pallas import tpu as pltpu

N = 100000
DIM_H = 512
NUM_GRAPHS = 256
ROWS = 5000
NB = N // ROWS


def _fused_body(batch_ref, h_ref, W1_ref, b1_ref, W2_ref, b2_ref,
                out_ref, acc_ref, vn_ref):
    i = pl.program_id(0)

    @pl.when(i == 0)
    def _init():
        acc_ref[...] = jnp.zeros_like(acc_ref)

    @pl.when(i < NB)
    def _pool():
        ids = batch_ref[0, 0, :]
        seg = lax.broadcasted_iota(jnp.int32, (NUM_GRAPHS, ROWS), 0)
        onehot = (ids[None, :] == seg).astype(jnp.float32)
        acc_ref[...] += jnp.dot(onehot, h_ref[...],
                                preferred_element_type=jnp.float32)

    @pl.when(i == NB - 1)
    def _ffn():
        s = acc_ref[...]
        z = jnp.maximum(jnp.dot(s, W1_ref[...],
                                preferred_element_type=jnp.float32)
                        + b1_ref[...], 0.0)
        vn_ref[...] = jnp.dot(z, W2_ref[...],
                              preferred_element_type=jnp.float32) + b2_ref[...]

    @pl.when(i >= NB)
    def _broadcast():
        ids = batch_ref[0, 0, :]
        seg = lax.broadcasted_iota(jnp.int32, (ROWS, NUM_GRAPHS), 1)
        onehot = (ids[:, None] == seg).astype(jnp.float32)
        out_ref[...] = jnp.dot(onehot, vn_ref[...],
                               preferred_element_type=jnp.float32)


@jax.jit
def kernel(h, batch, W1, b1, W2, b2):
    batch3 = batch.astype(jnp.int32).reshape(NB, 1, ROWS)

    out = pl.pallas_call(
        _fused_body,
        grid=(2 * NB,),
        in_specs=[
            pl.BlockSpec((1, 1, ROWS),
                         lambda i: (jnp.where(i < NB, i, i - NB), 0, 0)),
            pl.BlockSpec((ROWS, DIM_H),
                         lambda i: (jnp.minimum(i, NB - 1), 0)),
            pl.BlockSpec((DIM_H, 2 * DIM_H), lambda i: (0, 0)),
            pl.BlockSpec((2 * DIM_H,), lambda i: (0,)),
            pl.BlockSpec((2 * DIM_H, DIM_H), lambda i: (0, 0)),
            pl.BlockSpec((DIM_H,), lambda i: (0,)),
        ],
        out_specs=pl.BlockSpec((ROWS, DIM_H),
                               lambda i: (jnp.maximum(i - NB, 0), 0)),
        out_shape=jax.ShapeDtypeStruct((N, DIM_H), jnp.float32),
        scratch_shapes=[pltpu.VMEM((NUM_GRAPHS, DIM_H), jnp.float32),
                        pltpu.VMEM((NUM_GRAPHS, DIM_H), jnp.float32)],
    )(batch3, h, W1, b1, W2, b2)
    return out
